# baseline (speedup 1.0000x reference)
import dataclasses
import functools

import jax
import jax.numpy as jnp
from jax import lax
from jax.experimental import pallas as pl
from jax.experimental.pallas import tpu as pltpu
from jax.experimental.pallas import tpu_sc as plsc

B = 16384
F = 64
NC = 2
NS = 16
NW = NC * NS
BPW = B // NW
NCHUNK = BPW // 128


def _sc_predict(uids2d, iids2d, uf2, if2, ub16, ib16, gb16):
    mesh = plsc.VectorSubcoreMesh(core_axis_name="c", subcore_axis_name="s")
    cp = pltpu.CompilerParams()
    if "needs_layout_passes" in pltpu.CompilerParams.__dataclass_fields__:
        cp = dataclasses.replace(cp, needs_layout_passes=False)
    if "use_tc_tiling_on_sc" in pltpu.CompilerParams.__dataclass_fields__:
        cp = dataclasses.replace(cp, use_tc_tiling_on_sc=False)

    @functools.partial(
        pl.kernel,
        mesh=mesh,
        compiler_params=cp,
        out_type=jax.ShapeDtypeStruct((B,), jnp.float32),
        scratch_types=[
            pltpu.VMEM((NCHUNK, 128), jnp.int32),
            pltpu.VMEM((NCHUNK, 128), jnp.int32),
            pltpu.VMEM((NCHUNK, 128), jnp.int32),
            pltpu.VMEM((NCHUNK, 128), jnp.int32),
            pltpu.VMEM((NCHUNK, 128), jnp.int32),
            pltpu.VMEM((NCHUNK, 128), jnp.int32),
            pltpu.VMEM((2, 128, 128), jnp.float32),
            pltpu.VMEM((2, 128, 128), jnp.float32),
            pltpu.VMEM((BPW, 16), jnp.float32),
            pltpu.VMEM((BPW, 16), jnp.float32),
            pltpu.VMEM((BPW + 16,), jnp.float32),
            pltpu.VMEM((BPW,), jnp.float32),
            pltpu.VMEM((16,), jnp.float32),
            pltpu.SemaphoreType.DMA,
            pltpu.SemaphoreType.DMA,
            pltpu.SemaphoreType.DMA,
        ],
    )
    def body(uids_hbm, iids_hbm, uf_hbm, if_hbm, ubias_hbm, ibias_hbm, gb_hbm,
             out_hbm, idx_u, idx_i, idx_pu, idx_pi, idx_su, idx_si, u2, i2,
             ub_g, ib_g, dots, out_v, gb_v, sem_b, sem_u, sem_i):
        wid = lax.axis_index("s") * NC + lax.axis_index("c")
        base = wid * BPW

        pltpu.sync_copy(uids_hbm.at[pl.ds(wid * NCHUNK, NCHUNK)], idx_u)
        pltpu.sync_copy(iids_hbm.at[pl.ds(wid * NCHUNK, NCHUNK)], idx_i)
        pltpu.sync_copy(gb_hbm, gb_v)

        for j in range(NCHUNK):
            for k in range(8):
                s = pl.ds(k * 16, 16)
                idx_pu[j, s] = lax.shift_right_logical(idx_u[j, s], 1)
                idx_pi[j, s] = lax.shift_right_logical(idx_i[j, s], 1)
                idx_su[j, s] = lax.shift_right_logical(idx_u[j, s], 4)
                idx_si[j, s] = lax.shift_right_logical(idx_i[j, s], 4)

        bias_handles = []
        for j in range(NCHUNK):
            dst = pl.ds(j * 128, 128)
            bias_handles.append(
                pltpu.async_copy(ubias_hbm.at[idx_su.at[j]], ub_g.at[dst],
                                 sem_b))
            bias_handles.append(
                pltpu.async_copy(ibias_hbm.at[idx_si.at[j]], ib_g.at[dst],
                                 sem_b))

        lane = lax.iota(jnp.int32, 16)
        last_lane = lane == 15

        def vgather(v, idx16):
            dnums = lax.GatherDimensionNumbers(
                offset_dims=(), collapsed_slice_dims=(0,), start_index_map=(0,))
            return lax.gather(v, idx16[:, None], dnums, (1,),
                              mode=lax.GatherScatterMode.PROMISE_IN_BOUNDS)

        def fire(j):
            buf = j % 2
            return (pltpu.async_copy(uf_hbm.at[idx_pu.at[j]], u2.at[buf],
                                     sem_u),
                    pltpu.async_copy(if_hbm.at[idx_pi.at[j]], i2.at[buf],
                                     sem_i))

        def compute_chunk(j):
            buf = j % 2
            ub = u2.at[buf]
            ib = i2.at[buf]

            @pl.loop(0, 128, step=16)
            def _(g16, j=j, ub=ub, ib=ib):
                gs = pl.ds(g16, 16)
                half_u = idx_u[j, gs] & 1
                half_i = idx_i[j, gs] & 1
                for l in range(16):
                    sel = jnp.full((16,), l, jnp.int32)
                    mu = vgather(half_u, sel) != 0
                    mi = vgather(half_i, sel) != 0
                    r = g16 + l
                    p = None
                    for c in range(F // 16):
                        ulo = ub[r, pl.ds(c * 16, 16)]
                        uhi = ub[r, pl.ds(64 + c * 16, 16)]
                        ilo = ib[r, pl.ds(c * 16, 16)]
                        ihi = ib[r, pl.ds(64 + c * 16, 16)]
                        us = jnp.where(mu, uhi, ulo)
                        is_ = jnp.where(mi, ihi, ilo)
                        p = us * is_ if p is None else p + us * is_
                    cs = plsc.cumsum(p)
                    plsc.store_compressed(dots.at[pl.ds(j * 128 + r, 16)], cs,
                                          mask=last_lane)

        handles = fire(0)
        for j in range(NCHUNK):
            for h in handles:
                h.wait()
            if j + 1 < NCHUNK:
                handles = fire(j + 1)
            compute_chunk(j)

        for h in bias_handles:
            h.wait()

        gb_vec = gb_v[...]

        for j in range(NCHUNK):
            @pl.loop(0, 128, step=16)
            def _(off, j=j):
                c = j * 128 + off
                d = dots[pl.ds(c, 16)]
                row = lane + c
                mod_u = idx_u[j, pl.ds(off, 16)] & 15
                mod_i = idx_i[j, pl.ds(off, 16)] & 15
                ubv = plsc.load_gather(ub_g, [row, mod_u])
                ibv = plsc.load_gather(ib_g, [row, mod_i])
                pred = d + ubv + ibv + gb_vec
                pred = jnp.minimum(jnp.maximum(pred, 1.0), 5.0)
                out_v[pl.ds(c, 16)] = pred

        pltpu.sync_copy(out_v, out_hbm.at[pl.ds(base, BPW)])

    return body(uids2d, iids2d, uf2, if2, ub16, ib16, gb16)


def kernel(user_ids, item_ids, user_factors, item_factors, user_biases,
           item_biases, global_bias):
    uids2d = user_ids.reshape(NW * NCHUNK, 128)
    iids2d = item_ids.reshape(NW * NCHUNK, 128)
    uf2 = user_factors.reshape(-1, 128)
    if2 = item_factors.reshape(-1, 128)
    ub16 = user_biases.reshape(-1, 16)
    ib16 = item_biases.reshape(-1, 16)
    gb16 = jnp.broadcast_to(global_bias.astype(jnp.float32), (16,))
    return _sc_predict(uids2d, iids2d, uf2, if2, ub16, ib16, gb16)
